# baseline (speedup 1.0000x reference)
import functools
import jax
import jax.numpy as jnp
from jax import lax
from jax.experimental import pallas as pl
from jax.experimental.pallas import tpu as pltpu
from jax.experimental.pallas import tpu_sc as plsc

N_NODES = 10000
D = 128
N_EDGES = 320000
ALPHA = 0.2

NC = 2
NS = 16
L = 16
NW = NC * NS
EPW = N_EDGES // NW
CHUNK = 40
NCHUNKS = EPW // CHUNK
FG = 2
TE = CHUNK - FG * L
ROWS_PT = 624
TAIL_ROWS = N_NODES - NS * ROWS_PT
ZROWS = 8
NB = 5
P1D = NB // 2


def _sc_body(emb_hbm, srcw_hbm, dstw_hbm,
             parts_hbm, z_hbm, m_hbm, sco_hbm,
             isa, ida, bufs_l, sib, tbufs, scstg,
             vec16, mmat,
             acc_sh, max_sh,
             sg_l, ss_l, sw_l, sr_l):
    c = lax.axis_index("c")
    s = lax.axis_index("s")
    wid = c * NS + s

    bufs = tuple(bufs_l)
    sidx = tuple(sib.at[k] for k in range(NB))
    sg = tuple(sg_l)
    ss = tuple(ss_l)
    sw = tuple(sw_l)
    sr = tuple(sr_l)

    lanes = lax.iota(jnp.int32, L)
    l16 = lanes * L
    zero_v = jnp.zeros((L,), jnp.float32)
    ninf_v = jnp.full((L,), -jnp.inf, jnp.float32)
    tmask = lanes < TE

    ebase = wid * EPW
    pltpu.sync_copy(srcw_hbm.at[pl.ds(ebase, EPW)], isa)
    pltpu.sync_copy(dstw_hbm.at[pl.ds(ebase, EPW)], ida)

    def issue1(ci, b):
        pltpu.async_copy(emb_hbm.at[isa.at[pl.ds(ci * CHUNK, CHUNK)]],
                         bufs[b], sg[b])
        pltpu.async_copy(emb_hbm.at[ida.at[pl.ds(ci * CHUNK, CHUNK)]],
                         bufs[P1D + b], sg[P1D + b])

    def issue2(ci, b):
        pltpu.async_copy(emb_hbm.at[ida.at[pl.ds(ci * CHUNK, CHUNK)]],
                         bufs[b], sg[b])
        pltpu.async_copy(sco_hbm.at[pl.ds(ebase + ci * CHUNK, CHUNK)],
                         scstg.at[b, pl.ds(0, CHUNK)], sr[b])

    def wait_g(k):
        pltpu.make_async_copy(emb_hbm.at[isa.at[pl.ds(0, CHUNK)]],
                              bufs[k], sg[k]).wait()

    def wait_s(k):
        pltpu.make_async_copy(bufs[k], acc_sh.at[sidx[k]], ss[k]).wait()

    def wait_sw(k):
        pltpu.make_async_copy(scstg.at[k, pl.ds(0, CHUNK)],
                              sco_hbm.at[pl.ds(0, CHUNK)], sw[k]).wait()

    def wait_sr(k):
        pltpu.make_async_copy(sco_hbm.at[pl.ds(0, CHUNK)],
                              scstg.at[k, pl.ds(0, CHUNK)], sr[k]).wait()

    zb = bufs[NB - 1]

    def zloop(i, _):
        for k in range(D // L):
            zb[i, pl.ds(k * L, L)] = zero_v
        return 0
    lax.fori_loop(0, CHUNK, zloop, 0)
    for b in range(ROWS_PT // CHUNK):
        pltpu.sync_copy(
            zb, acc_sh.at[pl.ds(s * ROWS_PT + b * CHUNK, CHUNK)])
    zrem = ROWS_PT % CHUNK
    pltpu.sync_copy(
        zb.at[pl.ds(0, zrem)],
        acc_sh.at[pl.ds(s * ROWS_PT + ROWS_PT - zrem, zrem)])

    @pl.when(s == NS - 1)
    def _zero_tail():
        pltpu.sync_copy(zb.at[pl.ds(0, TAIL_ROWS)],
                        acc_sh.at[pl.ds(NS * ROWS_PT, TAIL_ROWS)])

    for k in range(P1D):
        issue1(k, k)

    def ztb(i, _):
        tbufs[pl.ds(L * L + i * L, L)] = zero_v
        return 0
    lax.fori_loop(0, L, ztb, 0)

    def dots(bi, bj, tb_off, g, n_e):
        def edge(e, _):
            r = g * L + e
            acc = bi[r, pl.ds(0, L)] * bj[r, pl.ds(0, L)]
            for k in range(1, D // L):
                acc = acc + bi[r, pl.ds(k * L, L)] * bj[r, pl.ds(k * L, L)]
            tbufs[pl.ds(tb_off + e * L, L)] = acc
            return 0
        lax.fori_loop(0, n_e, edge, 0)
        svec = plsc.load_gather(tbufs, [l16 + tb_off])
        for k in range(1, L):
            svec = svec + plsc.load_gather(tbufs, [l16 + (tb_off + k)])
        return jnp.where(svec >= 0.0, svec, ALPHA * svec)

    def score_chunk(ci, b, lmax):
        bi = bufs[b]
        bj = bufs[P1D + b]
        for g in range(FG):
            svec = dots(bi, bj, 0, g, L)
            scstg[b, pl.ds(g * L, L)] = svec
            lmax = jnp.maximum(lmax, svec)
        svec = dots(bi, bj, L * L, FG, TE)
        scstg[b, pl.ds(FG * L, L)] = svec
        return jnp.maximum(lmax, jnp.where(tmask, svec, ninf_v))

    def score_out(ci, b):
        pltpu.async_copy(scstg.at[b, pl.ds(0, CHUNK)],
                         sco_hbm.at[pl.ds(ebase + ci * CHUNK, CHUNK)],
                         sw[b])

    def step1(ci, b, lmax):
        @pl.when(ci >= P1D)
        def _wait_prev_scw():
            wait_sw(b)
        wait_g(b)
        wait_g(P1D + b)
        lmax = score_chunk(ci, b, lmax)

        @pl.when(ci >= 1)
        def _flush_prev_scores():
            score_out(ci - 1, (b + P1D - 1) % P1D)

        @pl.when(ci + P1D < NCHUNKS)
        def _issue_next():
            issue1(ci + P1D, b)
        return lmax

    def grp1(p, lmax):
        for b in range(P1D):
            lmax = step1(P1D * p + b, b, lmax)
        return lmax
    lmax = lax.fori_loop(0, NCHUNKS // P1D, grp1,
                         jnp.full((L,), -jnp.inf, jnp.float32))
    for t in range(NCHUNKS % P1D):
        lmax = step1(NCHUNKS - (NCHUNKS % P1D) + t, t, lmax)
    score_out(NCHUNKS - 1, (NCHUNKS - 1) % P1D)
    for k in range(P1D):
        wait_sw(k)

    for b in range(NB - 1):
        issue2(b, b)

    vec16[...] = lmax
    pltpu.sync_copy(vec16, max_sh.at[pl.ds(s * L, L)])
    plsc.subcore_barrier()
    pltpu.sync_copy(max_sh, mmat)
    mv = mmat[pl.ds(0, L)]
    for rr in range(1, NS):
        mv = jnp.maximum(mv, mmat[pl.ds(rr * L, L)])
    m_sc = jnp.max(mv)

    def scale_chunk(b, zacc):
        bj = bufs[b]
        for g in range(FG + 1):
            sv = scstg[b, pl.ds(g * L, L)]
            p = jnp.exp(sv - m_sc)
            n_e = L if g < FG else TE
            for e in range(n_e):
                pr = p[e]
                r = g * L + e
                for k in range(D // L):
                    bj[r, pl.ds(k * L, L)] = bj[r, pl.ds(k * L, L)] * pr
            if g < FG:
                zacc = zacc + p
            else:
                zacc = zacc + jnp.where(tmask, p, zero_v)
        return zacc

    def build_sidx(ci, b):
        si = sidx[b]
        si[pl.ds(0, L)] = isa[pl.ds(ci * CHUNK, L)]
        si[pl.ds(L, L)] = isa[pl.ds(ci * CHUNK + L, L)]
        si[pl.ds(CHUNK - L, L)] = isa[pl.ds(ci * CHUNK + CHUNK - L, L)]

    def step2(ci, b, zacc):
        pb = (b + NB - 1) % NB
        wait_g(b)
        wait_sr(b)

        @pl.when(ci >= 1)
        def _wait_prev_scatter():
            wait_s(pb)

        @pl.when(ci + NB - 1 < NCHUNKS)
        def _issue_next():
            issue2(ci + NB - 1, pb)
        build_sidx(ci, b)
        zacc = scale_chunk(b, zacc)
        pltpu.async_copy(bufs[b], acc_sh.at[sidx[b]], ss[b], add=True)
        return zacc

    def grp2(q, zacc):
        for b in range(NB):
            zacc = step2(NB * q + b, b, zacc)
        return zacc
    zacc = lax.fori_loop(0, NCHUNKS // NB, grp2, jnp.zeros((L,), jnp.float32))
    for t in range(NCHUNKS % NB):
        zacc = step2(NCHUNKS - (NCHUNKS % NB) + t, t, zacc)
    wait_s((NCHUNKS - 1) % NB)

    vec16[...] = zacc
    pltpu.sync_copy(vec16, z_hbm.at[pl.ds(wid * L, L)])
    vec16[...] = jnp.full((L,), m_sc, jnp.float32)
    pltpu.sync_copy(vec16, m_hbm.at[pl.ds(wid * L, L)])

    plsc.subcore_barrier()
    pltpu.sync_copy(acc_sh.at[pl.ds(s * ROWS_PT, ROWS_PT)],
                    parts_hbm.at[c, pl.ds(s * ROWS_PT, ROWS_PT)])

    @pl.when(s == NS - 1)
    def _wb_tail():
        rb = NS * ROWS_PT
        pltpu.sync_copy(acc_sh.at[pl.ds(rb, TAIL_ROWS)],
                        parts_hbm.at[c, pl.ds(rb, TAIL_ROWS)])


@functools.partial(
    pl.kernel,
    out_type=(
        jax.ShapeDtypeStruct((NC, N_NODES, D), jnp.float32),
        jax.ShapeDtypeStruct((NW * L,), jnp.float32),
        jax.ShapeDtypeStruct((NW * L,), jnp.float32),
        jax.ShapeDtypeStruct((N_EDGES,), jnp.float32),
    ),
    mesh=plsc.VectorSubcoreMesh(core_axis_name="c", subcore_axis_name="s"),
    compiler_params=pltpu.CompilerParams(needs_layout_passes=False),
    scratch_types=(
        pltpu.VMEM_SHARED((N_NODES, D), jnp.float32),
        pltpu.VMEM_SHARED((NS * L,), jnp.float32),
        *([pltpu.SemaphoreType.DMA] * (3 * NB + NB // 2)),
    ),
)
def _sc_attention(emb_hbm, srcw_hbm, dstw_hbm, parts_hbm, z_hbm, m_hbm,
                  sco_hbm, *rest):
    def scoped(isa, ida, *r2):
        bufs_l = r2[:NB]
        sib, tbufs, scstg, vec16, mmat = r2[NB:]
        _sc_body(emb_hbm, srcw_hbm, dstw_hbm, parts_hbm, z_hbm, m_hbm,
                 sco_hbm,
                 isa, ida, bufs_l, sib, tbufs, scstg,
                 vec16, mmat, acc_sh, max_sh,
                 rest_sems[:NB], rest_sems[NB:2 * NB],
                 rest_sems[2 * NB:2 * NB + P1D], rest_sems[2 * NB + P1D:])
    acc_sh, max_sh = rest[0], rest[1]
    rest_sems = rest[2:]
    pl.run_scoped(
        scoped,
        pltpu.VMEM((EPW,), jnp.int32),
        pltpu.VMEM((EPW,), jnp.int32),
        *[pltpu.VMEM((CHUNK, D), jnp.float32) for _ in range(NB)],
        pltpu.VMEM((NB, CHUNK), jnp.int32),
        pltpu.VMEM((2 * L * L,), jnp.float32),
        pltpu.VMEM((NB, CHUNK + 8), jnp.float32),
        pltpu.VMEM((L,), jnp.float32),
        pltpu.VMEM((NS * L,), jnp.float32),
    )


def _combine_body(z_ref, m_ref, parts_ref, emb_ref, out_ref):
    half = NS * L
    m = m_ref[...]
    zv = z_ref[...]
    m0 = jnp.max(m[:half])
    m1 = jnp.max(m[half:])
    mg = jnp.maximum(m0, m1)
    a0 = jnp.exp(m0 - mg)
    a1 = jnp.exp(m1 - mg)
    z = jnp.sum(zv[:half]) * a0 + jnp.sum(zv[half:]) * a1
    inv = 1.0 / z
    out_ref[...] = (parts_ref[0] * a0 + parts_ref[1] * a1) * inv + emb_ref[...]


def _combine(parts, z, m, emb):
    grid = 10
    rows = N_NODES // grid
    return pl.pallas_call(
        _combine_body,
        grid=(grid,),
        in_specs=[
            pl.BlockSpec((NW * L,), lambda i: (0,)),
            pl.BlockSpec((NW * L,), lambda i: (0,)),
            pl.BlockSpec((NC, rows, D), lambda i: (0, i, 0)),
            pl.BlockSpec((rows, D), lambda i: (i, 0)),
        ],
        out_specs=pl.BlockSpec((rows, D), lambda i: (i, 0)),
        out_shape=jax.ShapeDtypeStruct((N_NODES, D), jnp.float32),
    )(z, m, parts, emb)


@jax.jit
def kernel(drug_embeddings, drug_relationships):
    er = drug_relationships.astype(jnp.int32)
    srcw = er[:, 0]
    dstw = er[:, 1]
    parts, z, m, _ = _sc_attention(drug_embeddings, srcw, dstw)
    return _combine(parts, z, m, drug_embeddings)


# speedup vs baseline: 1.0006x; 1.0006x over previous
import functools
import jax
import jax.numpy as jnp
from jax import lax
from jax.experimental import pallas as pl
from jax.experimental.pallas import tpu as pltpu
from jax.experimental.pallas import tpu_sc as plsc

N_NODES = 10000
D = 128
N_EDGES = 320000
ALPHA = 0.2

NC = 2
NS = 16
L = 16
NW = NC * NS
EPW = N_EDGES // NW
CHUNK = 40
NCHUNKS = EPW // CHUNK
FG = 2
TE = CHUNK - FG * L
ROWS_PT = 624
TAIL_ROWS = N_NODES - NS * ROWS_PT
ZROWS = 8
NB = 5
P1D = NB // 2


def _sc_body(emb_hbm, srcw_hbm, dstw_hbm,
             parts_hbm, z_hbm, m_hbm, sco_hbm,
             isa, ida, bufs_l, sib, tbufs, scstg,
             vec16, mmat,
             acc_sh, max_sh,
             sg_l, ss_l, sw_l, sr_l):
    c = lax.axis_index("c")
    s = lax.axis_index("s")
    wid = c * NS + s

    bufs = tuple(bufs_l)
    sidx = tuple(sib.at[k] for k in range(NB))
    sg = tuple(sg_l)
    ss = tuple(ss_l)
    sw = tuple(sw_l)
    sr = tuple(sr_l)

    lanes = lax.iota(jnp.int32, L)
    l16 = lanes * L
    zero_v = jnp.zeros((L,), jnp.float32)
    ninf_v = jnp.full((L,), -jnp.inf, jnp.float32)
    tmask = lanes < TE

    ebase = wid * EPW
    pltpu.sync_copy(srcw_hbm.at[pl.ds(ebase, EPW)], isa)
    pltpu.sync_copy(dstw_hbm.at[pl.ds(ebase, EPW)], ida)

    def issue1(ci, b):
        pltpu.async_copy(emb_hbm.at[isa.at[pl.ds(ci * CHUNK, CHUNK)]],
                         bufs[b], sg[b])
        pltpu.async_copy(emb_hbm.at[ida.at[pl.ds(ci * CHUNK, CHUNK)]],
                         bufs[P1D + b], sg[P1D + b])

    def issue2(ci, b):
        pltpu.async_copy(emb_hbm.at[ida.at[pl.ds(ci * CHUNK, CHUNK)]],
                         bufs[b], sg[b])
        pltpu.async_copy(sco_hbm.at[pl.ds(ebase + ci * CHUNK, CHUNK)],
                         scstg.at[b, pl.ds(0, CHUNK)], sr[b])

    def wait_g(k):
        pltpu.make_async_copy(emb_hbm.at[isa.at[pl.ds(0, CHUNK)]],
                              bufs[k], sg[k]).wait()

    def wait_s(k):
        pltpu.make_async_copy(bufs[k], acc_sh.at[sidx[k]], ss[k]).wait()

    def wait_sw(k):
        pltpu.make_async_copy(scstg.at[k, pl.ds(0, CHUNK)],
                              sco_hbm.at[pl.ds(0, CHUNK)], sw[k]).wait()

    def wait_sr(k):
        pltpu.make_async_copy(sco_hbm.at[pl.ds(0, CHUNK)],
                              scstg.at[k, pl.ds(0, CHUNK)], sr[k]).wait()

    for k in range(P1D):
        issue1(k, k)

    zb = bufs[NB - 1]

    def zloop(i, _):
        for k in range(D // L):
            zb[i, pl.ds(k * L, L)] = zero_v
        return 0
    lax.fori_loop(0, CHUNK, zloop, 0)
    for b in range(ROWS_PT // CHUNK):
        pltpu.sync_copy(
            zb, acc_sh.at[pl.ds(s * ROWS_PT + b * CHUNK, CHUNK)])
    zrem = ROWS_PT % CHUNK
    pltpu.sync_copy(
        zb.at[pl.ds(0, zrem)],
        acc_sh.at[pl.ds(s * ROWS_PT + ROWS_PT - zrem, zrem)])

    @pl.when(s == NS - 1)
    def _zero_tail():
        pltpu.sync_copy(zb.at[pl.ds(0, TAIL_ROWS)],
                        acc_sh.at[pl.ds(NS * ROWS_PT, TAIL_ROWS)])

    def ztb(i, _):
        tbufs[pl.ds(L * L + i * L, L)] = zero_v
        return 0
    lax.fori_loop(0, L, ztb, 0)

    def dots(bi, bj, tb_off, g, n_e):
        def edge(e, _):
            r = g * L + e
            acc = bi[r, pl.ds(0, L)] * bj[r, pl.ds(0, L)]
            for k in range(1, D // L):
                acc = acc + bi[r, pl.ds(k * L, L)] * bj[r, pl.ds(k * L, L)]
            tbufs[pl.ds(tb_off + e * L, L)] = acc
            return 0
        lax.fori_loop(0, n_e, edge, 0)
        svec = plsc.load_gather(tbufs, [l16 + tb_off])
        for k in range(1, L):
            svec = svec + plsc.load_gather(tbufs, [l16 + (tb_off + k)])
        return jnp.where(svec >= 0.0, svec, ALPHA * svec)

    def score_chunk(ci, b, lmax):
        bi = bufs[b]
        bj = bufs[P1D + b]
        for g in range(FG):
            svec = dots(bi, bj, 0, g, L)
            scstg[b, pl.ds(g * L, L)] = svec
            lmax = jnp.maximum(lmax, svec)
        svec = dots(bi, bj, L * L, FG, TE)
        scstg[b, pl.ds(FG * L, L)] = svec
        return jnp.maximum(lmax, jnp.where(tmask, svec, ninf_v))

    def score_out(ci, b):
        pltpu.async_copy(scstg.at[b, pl.ds(0, CHUNK)],
                         sco_hbm.at[pl.ds(ebase + ci * CHUNK, CHUNK)],
                         sw[b])

    def step1(ci, b, lmax):
        @pl.when(ci >= P1D)
        def _wait_prev_scw():
            wait_sw(b)
        wait_g(b)
        wait_g(P1D + b)
        lmax = score_chunk(ci, b, lmax)

        @pl.when(ci >= 1)
        def _flush_prev_scores():
            score_out(ci - 1, (b + P1D - 1) % P1D)

        @pl.when(ci + P1D < NCHUNKS)
        def _issue_next():
            issue1(ci + P1D, b)
        return lmax

    def grp1(p, lmax):
        for b in range(P1D):
            lmax = step1(P1D * p + b, b, lmax)
        return lmax
    lmax = lax.fori_loop(0, NCHUNKS // P1D, grp1,
                         jnp.full((L,), -jnp.inf, jnp.float32))
    for t in range(NCHUNKS % P1D):
        lmax = step1(NCHUNKS - (NCHUNKS % P1D) + t, t, lmax)
    score_out(NCHUNKS - 1, (NCHUNKS - 1) % P1D)
    for k in range(P1D):
        wait_sw(k)

    for b in range(NB - 1):
        issue2(b, b)

    vec16[...] = lmax
    pltpu.sync_copy(vec16, max_sh.at[pl.ds(s * L, L)])
    plsc.subcore_barrier()
    pltpu.sync_copy(max_sh, mmat)
    mv = mmat[pl.ds(0, L)]
    for rr in range(1, NS):
        mv = jnp.maximum(mv, mmat[pl.ds(rr * L, L)])
    m_sc = jnp.max(mv)

    def scale_chunk(b, zacc):
        bj = bufs[b]
        for g in range(FG + 1):
            sv = scstg[b, pl.ds(g * L, L)]
            p = jnp.exp(sv - m_sc)
            n_e = L if g < FG else TE
            for e in range(n_e):
                pr = p[e]
                r = g * L + e
                for k in range(D // L):
                    bj[r, pl.ds(k * L, L)] = bj[r, pl.ds(k * L, L)] * pr
            if g < FG:
                zacc = zacc + p
            else:
                zacc = zacc + jnp.where(tmask, p, zero_v)
        return zacc

    def build_sidx(ci, b):
        si = sidx[b]
        si[pl.ds(0, L)] = isa[pl.ds(ci * CHUNK, L)]
        si[pl.ds(L, L)] = isa[pl.ds(ci * CHUNK + L, L)]
        si[pl.ds(CHUNK - L, L)] = isa[pl.ds(ci * CHUNK + CHUNK - L, L)]

    def step2(ci, b, zacc):
        pb = (b + NB - 1) % NB
        wait_g(b)
        wait_sr(b)

        @pl.when(ci >= 1)
        def _wait_prev_scatter():
            wait_s(pb)

        @pl.when(ci + NB - 1 < NCHUNKS)
        def _issue_next():
            issue2(ci + NB - 1, pb)
        build_sidx(ci, b)
        zacc = scale_chunk(b, zacc)
        pltpu.async_copy(bufs[b], acc_sh.at[sidx[b]], ss[b], add=True)
        return zacc

    def grp2(q, zacc):
        for b in range(NB):
            zacc = step2(NB * q + b, b, zacc)
        return zacc
    zacc = lax.fori_loop(0, NCHUNKS // NB, grp2, jnp.zeros((L,), jnp.float32))
    for t in range(NCHUNKS % NB):
        zacc = step2(NCHUNKS - (NCHUNKS % NB) + t, t, zacc)
    wait_s((NCHUNKS - 1) % NB)

    vec16[...] = zacc
    pltpu.sync_copy(vec16, z_hbm.at[pl.ds(wid * L, L)])
    vec16[...] = jnp.full((L,), m_sc, jnp.float32)
    pltpu.sync_copy(vec16, m_hbm.at[pl.ds(wid * L, L)])

    plsc.subcore_barrier()
    pltpu.sync_copy(acc_sh.at[pl.ds(s * ROWS_PT, ROWS_PT)],
                    parts_hbm.at[c, pl.ds(s * ROWS_PT, ROWS_PT)])

    @pl.when(s == NS - 1)
    def _wb_tail():
        rb = NS * ROWS_PT
        pltpu.sync_copy(acc_sh.at[pl.ds(rb, TAIL_ROWS)],
                        parts_hbm.at[c, pl.ds(rb, TAIL_ROWS)])


@functools.partial(
    pl.kernel,
    out_type=(
        jax.ShapeDtypeStruct((NC, N_NODES, D), jnp.float32),
        jax.ShapeDtypeStruct((NW * L,), jnp.float32),
        jax.ShapeDtypeStruct((NW * L,), jnp.float32),
        jax.ShapeDtypeStruct((N_EDGES,), jnp.float32),
    ),
    mesh=plsc.VectorSubcoreMesh(core_axis_name="c", subcore_axis_name="s"),
    compiler_params=pltpu.CompilerParams(needs_layout_passes=False),
    scratch_types=(
        pltpu.VMEM_SHARED((N_NODES, D), jnp.float32),
        pltpu.VMEM_SHARED((NS * L,), jnp.float32),
        *([pltpu.SemaphoreType.DMA] * (3 * NB + NB // 2)),
    ),
)
def _sc_attention(emb_hbm, srcw_hbm, dstw_hbm, parts_hbm, z_hbm, m_hbm,
                  sco_hbm, *rest):
    def scoped(isa, ida, *r2):
        bufs_l = r2[:NB]
        sib, tbufs, scstg, vec16, mmat = r2[NB:]
        _sc_body(emb_hbm, srcw_hbm, dstw_hbm, parts_hbm, z_hbm, m_hbm,
                 sco_hbm,
                 isa, ida, bufs_l, sib, tbufs, scstg,
                 vec16, mmat, acc_sh, max_sh,
                 rest_sems[:NB], rest_sems[NB:2 * NB],
                 rest_sems[2 * NB:2 * NB + P1D], rest_sems[2 * NB + P1D:])
    acc_sh, max_sh = rest[0], rest[1]
    rest_sems = rest[2:]
    pl.run_scoped(
        scoped,
        pltpu.VMEM((EPW,), jnp.int32),
        pltpu.VMEM((EPW,), jnp.int32),
        *[pltpu.VMEM((CHUNK, D), jnp.float32) for _ in range(NB)],
        pltpu.VMEM((NB, CHUNK), jnp.int32),
        pltpu.VMEM((2 * L * L,), jnp.float32),
        pltpu.VMEM((NB, CHUNK + 8), jnp.float32),
        pltpu.VMEM((L,), jnp.float32),
        pltpu.VMEM((NS * L,), jnp.float32),
    )


def _combine_body(z_ref, m_ref, parts_ref, emb_ref, out_ref):
    half = NS * L
    m = m_ref[...]
    zv = z_ref[...]
    m0 = jnp.max(m[:half])
    m1 = jnp.max(m[half:])
    mg = jnp.maximum(m0, m1)
    a0 = jnp.exp(m0 - mg)
    a1 = jnp.exp(m1 - mg)
    z = jnp.sum(zv[:half]) * a0 + jnp.sum(zv[half:]) * a1
    inv = 1.0 / z
    out_ref[...] = (parts_ref[0] * a0 + parts_ref[1] * a1) * inv + emb_ref[...]


def _combine(parts, z, m, emb):
    grid = 10
    rows = N_NODES // grid
    return pl.pallas_call(
        _combine_body,
        grid=(grid,),
        in_specs=[
            pl.BlockSpec((NW * L,), lambda i: (0,)),
            pl.BlockSpec((NW * L,), lambda i: (0,)),
            pl.BlockSpec((NC, rows, D), lambda i: (0, i, 0)),
            pl.BlockSpec((rows, D), lambda i: (i, 0)),
        ],
        out_specs=pl.BlockSpec((rows, D), lambda i: (i, 0)),
        out_shape=jax.ShapeDtypeStruct((N_NODES, D), jnp.float32),
    )(z, m, parts, emb)


@jax.jit
def kernel(drug_embeddings, drug_relationships):
    er = drug_relationships.astype(jnp.int32)
    srcw = er[:, 0]
    dstw = er[:, 1]
    parts, z, m, _ = _sc_attention(drug_embeddings, srcw, dstw)
    return _combine(parts, z, m, drug_embeddings)


# speedup vs baseline: 1.0372x; 1.0365x over previous
import functools
import jax
import jax.numpy as jnp
from jax import lax
from jax.experimental import pallas as pl
from jax.experimental.pallas import tpu as pltpu
from jax.experimental.pallas import tpu_sc as plsc

N_NODES = 10000
D = 128
N_EDGES = 320000
ALPHA = 0.2

NC = 2
NS = 16
L = 16
NW = NC * NS
EPW = N_EDGES // NW
CHUNK = 40
NCHUNKS = EPW // CHUNK
FG = 2
TE = CHUNK - FG * L
ROWS_PT = 624
TAIL_ROWS = N_NODES - NS * ROWS_PT
ZROWS = 16
NB = 4
P1D = NB // 2


def _sc_body(emb_hbm, srcw_hbm, dstw_hbm,
             parts_hbm, z_hbm, m_hbm, sco_hbm,
             isa, ida, bufs_l, sib, tbufs, scstg, stage,
             vec16, mmat,
             acc_sh, max_sh,
             sg_l, ss_l, sw_l, sr_l):
    c = lax.axis_index("c")
    s = lax.axis_index("s")
    wid = c * NS + s

    bufs = tuple(bufs_l)
    sidx = tuple(sib.at[k] for k in range(NB))
    sg = tuple(sg_l)
    ss = tuple(ss_l)
    sw = tuple(sw_l)
    sr = tuple(sr_l)

    lanes = lax.iota(jnp.int32, L)
    l16 = lanes * L
    zero_v = jnp.zeros((L,), jnp.float32)
    ninf_v = jnp.full((L,), -jnp.inf, jnp.float32)
    tmask = lanes < TE

    ebase = wid * EPW
    pltpu.sync_copy(srcw_hbm.at[pl.ds(ebase, EPW)], isa)
    pltpu.sync_copy(dstw_hbm.at[pl.ds(ebase, EPW)], ida)

    def issue1(ci, b):
        pltpu.async_copy(emb_hbm.at[isa.at[pl.ds(ci * CHUNK, CHUNK)]],
                         bufs[b], sg[b])
        pltpu.async_copy(emb_hbm.at[ida.at[pl.ds(ci * CHUNK, CHUNK)]],
                         bufs[P1D + b], sg[P1D + b])

    def issue2(ci, b):
        pltpu.async_copy(emb_hbm.at[ida.at[pl.ds(ci * CHUNK, CHUNK)]],
                         bufs[b], sg[b])
        pltpu.async_copy(sco_hbm.at[pl.ds(ebase + ci * CHUNK, CHUNK)],
                         scstg.at[b, pl.ds(0, CHUNK)], sr[b])

    def wait_g(k):
        pltpu.make_async_copy(emb_hbm.at[isa.at[pl.ds(0, CHUNK)]],
                              bufs[k], sg[k]).wait()

    def wait_s(k):
        pltpu.make_async_copy(bufs[k], acc_sh.at[sidx[k]], ss[k]).wait()

    def wait_sw(k):
        pltpu.make_async_copy(scstg.at[k, pl.ds(0, CHUNK)],
                              sco_hbm.at[pl.ds(0, CHUNK)], sw[k]).wait()

    def wait_sr(k):
        pltpu.make_async_copy(sco_hbm.at[pl.ds(0, CHUNK)],
                              scstg.at[k, pl.ds(0, CHUNK)], sr[k]).wait()

    for k in range(P1D):
        issue1(k, k)

    def zloop(i, _):
        for k in range(D // L):
            stage[i, pl.ds(k * L, L)] = zero_v
        return 0
    lax.fori_loop(0, ZROWS, zloop, 0)
    for b in range(ROWS_PT // ZROWS):
        pltpu.sync_copy(
            stage, acc_sh.at[pl.ds(s * ROWS_PT + b * ZROWS, ZROWS)])

    @pl.when(s == NS - 1)
    def _zero_tail():
        pltpu.sync_copy(stage,
                        acc_sh.at[pl.ds(NS * ROWS_PT, TAIL_ROWS)])

    def ztb(i, _):
        tbufs[pl.ds(L * L + i * L, L)] = zero_v
        return 0
    lax.fori_loop(0, L, ztb, 0)

    def dots(bi, bj, tb_off, g, n_e):
        def edge(e, _):
            r = g * L + e
            acc = bi[r, pl.ds(0, L)] * bj[r, pl.ds(0, L)]
            for k in range(1, D // L):
                acc = acc + bi[r, pl.ds(k * L, L)] * bj[r, pl.ds(k * L, L)]
            tbufs[pl.ds(tb_off + e * L, L)] = acc
            return 0
        lax.fori_loop(0, n_e, edge, 0)
        svec = plsc.load_gather(tbufs, [l16 + tb_off])
        for k in range(1, L):
            svec = svec + plsc.load_gather(tbufs, [l16 + (tb_off + k)])
        return jnp.where(svec >= 0.0, svec, ALPHA * svec)

    def score_chunk(ci, b, lmax):
        bi = bufs[b]
        bj = bufs[P1D + b]
        for g in range(FG):
            svec = dots(bi, bj, 0, g, L)
            scstg[b, pl.ds(g * L, L)] = svec
            lmax = jnp.maximum(lmax, svec)
        svec = dots(bi, bj, L * L, FG, TE)
        scstg[b, pl.ds(FG * L, L)] = svec
        return jnp.maximum(lmax, jnp.where(tmask, svec, ninf_v))

    def score_out(ci, b):
        pltpu.async_copy(scstg.at[b, pl.ds(0, CHUNK)],
                         sco_hbm.at[pl.ds(ebase + ci * CHUNK, CHUNK)],
                         sw[b])

    def step1(ci, b, lmax):
        @pl.when(ci >= P1D)
        def _wait_prev_scw():
            wait_sw(b)
        wait_g(b)
        wait_g(P1D + b)
        lmax = score_chunk(ci, b, lmax)

        @pl.when(ci >= 1)
        def _flush_prev_scores():
            score_out(ci - 1, (b + P1D - 1) % P1D)

        @pl.when(ci + P1D < NCHUNKS)
        def _issue_next():
            issue1(ci + P1D, b)
        return lmax

    def grp1(p, lmax):
        for b in range(P1D):
            lmax = step1(P1D * p + b, b, lmax)
        return lmax
    lmax = lax.fori_loop(0, NCHUNKS // P1D, grp1,
                         jnp.full((L,), -jnp.inf, jnp.float32))
    for t in range(NCHUNKS % P1D):
        lmax = step1(NCHUNKS - (NCHUNKS % P1D) + t, t, lmax)
    score_out(NCHUNKS - 1, (NCHUNKS - 1) % P1D)
    for k in range(P1D):
        wait_sw(k)

    for b in range(NB - 1):
        issue2(b, b)

    vec16[...] = lmax
    pltpu.sync_copy(vec16, max_sh.at[pl.ds(s * L, L)])
    plsc.subcore_barrier()
    pltpu.sync_copy(max_sh, mmat)
    mv = mmat[pl.ds(0, L)]
    for rr in range(1, NS):
        mv = jnp.maximum(mv, mmat[pl.ds(rr * L, L)])
    m_sc = jnp.max(mv)

    def scale_chunk(b, zacc):
        bj = bufs[b]
        for g in range(FG + 1):
            sv = scstg[b, pl.ds(g * L, L)]
            p = jnp.exp(sv - m_sc)
            n_e = L if g < FG else TE
            for e in range(n_e):
                pr = p[e]
                r = g * L + e
                for k in range(D // L):
                    bj[r, pl.ds(k * L, L)] = bj[r, pl.ds(k * L, L)] * pr
            if g < FG:
                zacc = zacc + p
            else:
                zacc = zacc + jnp.where(tmask, p, zero_v)
        return zacc

    def build_sidx(ci, b):
        si = sidx[b]
        si[pl.ds(0, L)] = isa[pl.ds(ci * CHUNK, L)]
        si[pl.ds(L, L)] = isa[pl.ds(ci * CHUNK + L, L)]
        si[pl.ds(CHUNK - L, L)] = isa[pl.ds(ci * CHUNK + CHUNK - L, L)]

    def step2(ci, b, zacc):
        pb = (b + NB - 1) % NB
        wait_g(b)
        wait_sr(b)

        @pl.when(ci >= 1)
        def _wait_prev_scatter():
            wait_s(pb)

        @pl.when(ci + NB - 1 < NCHUNKS)
        def _issue_next():
            issue2(ci + NB - 1, pb)
        build_sidx(ci, b)
        zacc = scale_chunk(b, zacc)
        pltpu.async_copy(bufs[b], acc_sh.at[sidx[b]], ss[b], add=True)
        return zacc

    def grp2(q, zacc):
        for b in range(NB):
            zacc = step2(NB * q + b, b, zacc)
        return zacc
    zacc = lax.fori_loop(0, NCHUNKS // NB, grp2, jnp.zeros((L,), jnp.float32))
    for t in range(NCHUNKS % NB):
        zacc = step2(NCHUNKS - (NCHUNKS % NB) + t, t, zacc)
    wait_s((NCHUNKS - 1) % NB)

    vec16[...] = zacc
    pltpu.sync_copy(vec16, z_hbm.at[pl.ds(wid * L, L)])
    vec16[...] = jnp.full((L,), m_sc, jnp.float32)
    pltpu.sync_copy(vec16, m_hbm.at[pl.ds(wid * L, L)])

    plsc.subcore_barrier()
    pltpu.sync_copy(acc_sh.at[pl.ds(s * ROWS_PT, ROWS_PT)],
                    parts_hbm.at[c, pl.ds(s * ROWS_PT, ROWS_PT)])

    @pl.when(s == NS - 1)
    def _wb_tail():
        rb = NS * ROWS_PT
        pltpu.sync_copy(acc_sh.at[pl.ds(rb, TAIL_ROWS)],
                        parts_hbm.at[c, pl.ds(rb, TAIL_ROWS)])


@functools.partial(
    pl.kernel,
    out_type=(
        jax.ShapeDtypeStruct((NC, N_NODES, D), jnp.float32),
        jax.ShapeDtypeStruct((NW * L,), jnp.float32),
        jax.ShapeDtypeStruct((NW * L,), jnp.float32),
        jax.ShapeDtypeStruct((N_EDGES,), jnp.float32),
    ),
    mesh=plsc.VectorSubcoreMesh(core_axis_name="c", subcore_axis_name="s"),
    compiler_params=pltpu.CompilerParams(needs_layout_passes=False),
    scratch_types=(
        pltpu.VMEM_SHARED((N_NODES, D), jnp.float32),
        pltpu.VMEM_SHARED((NS * L,), jnp.float32),
        *([pltpu.SemaphoreType.DMA] * (3 * NB + NB // 2)),
    ),
)
def _sc_attention(emb_hbm, srcw_hbm, dstw_hbm, parts_hbm, z_hbm, m_hbm,
                  sco_hbm, *rest):
    def scoped(isa, ida, *r2):
        bufs_l = r2[:NB]
        sib, tbufs, scstg, stage, vec16, mmat = r2[NB:]
        _sc_body(emb_hbm, srcw_hbm, dstw_hbm, parts_hbm, z_hbm, m_hbm,
                 sco_hbm,
                 isa, ida, bufs_l, sib, tbufs, scstg, stage,
                 vec16, mmat, acc_sh, max_sh,
                 rest_sems[:NB], rest_sems[NB:2 * NB],
                 rest_sems[2 * NB:2 * NB + P1D], rest_sems[2 * NB + P1D:])
    acc_sh, max_sh = rest[0], rest[1]
    rest_sems = rest[2:]
    pl.run_scoped(
        scoped,
        pltpu.VMEM((EPW,), jnp.int32),
        pltpu.VMEM((EPW,), jnp.int32),
        *[pltpu.VMEM((CHUNK, D), jnp.float32) for _ in range(NB)],
        pltpu.VMEM((NB, CHUNK), jnp.int32),
        pltpu.VMEM((2 * L * L,), jnp.float32),
        pltpu.VMEM((NB, CHUNK + 8), jnp.float32),
        pltpu.VMEM((ZROWS, D), jnp.float32),
        pltpu.VMEM((L,), jnp.float32),
        pltpu.VMEM((NS * L,), jnp.float32),
    )


def _combine_body(z_ref, m_ref, parts_ref, emb_ref, out_ref):
    half = NS * L
    m = m_ref[...]
    zv = z_ref[...]
    m0 = jnp.max(m[:half])
    m1 = jnp.max(m[half:])
    mg = jnp.maximum(m0, m1)
    a0 = jnp.exp(m0 - mg)
    a1 = jnp.exp(m1 - mg)
    z = jnp.sum(zv[:half]) * a0 + jnp.sum(zv[half:]) * a1
    inv = 1.0 / z
    out_ref[...] = (parts_ref[0] * a0 + parts_ref[1] * a1) * inv + emb_ref[...]


def _combine(parts, z, m, emb):
    grid = 10
    rows = N_NODES // grid
    return pl.pallas_call(
        _combine_body,
        grid=(grid,),
        in_specs=[
            pl.BlockSpec((NW * L,), lambda i: (0,)),
            pl.BlockSpec((NW * L,), lambda i: (0,)),
            pl.BlockSpec((NC, rows, D), lambda i: (0, i, 0)),
            pl.BlockSpec((rows, D), lambda i: (i, 0)),
        ],
        out_specs=pl.BlockSpec((rows, D), lambda i: (i, 0)),
        out_shape=jax.ShapeDtypeStruct((N_NODES, D), jnp.float32),
    )(z, m, parts, emb)


@jax.jit
def kernel(drug_embeddings, drug_relationships):
    er = drug_relationships.astype(jnp.int32)
    srcw = er[:, 0]
    dstw = er[:, 1]
    parts, z, m, _ = _sc_attention(drug_embeddings, srcw, dstw)
    return _combine(parts, z, m, drug_embeddings)
